# baseline (speedup 1.0000x reference)
import jax
import jax.numpy as jnp
from jax import lax
from jax.experimental import pallas as pl
from jax.experimental.pallas import tpu as pltpu
from jax.experimental.pallas import tpu_sc as plsc

B = 128
C = 32768
N_PAD = 1024

NC = 2
NW = 32
ROWS_PER_W = B // NW
NVEC = C // 16


def _sc_scale_body(feature_hbm, ks_hbm, out_hbm,
                   row_v, cnt, vsum, sufc, sufs, ksv, scv):
    wid = lax.axis_index("s") * NC + lax.axis_index("c")
    lanes = lax.iota(jnp.int32, 16)
    ones_i = jnp.ones((16,), jnp.int32)
    zero_i = jnp.zeros((16,), jnp.int32)
    zero_f = jnp.zeros((16,), jnp.float32)
    pltpu.sync_copy(ks_hbm.at[wid], ksv)
    ks_all = ksv[...]

    def row_body(i, _):
        r = wid * ROWS_PER_W + i
        pltpu.sync_copy(feature_hbm.at[r], row_v)
        ks = jnp.sum(jnp.where(lanes == i, ks_all, jnp.int32(0)))

        def zero_body(e, unused):
            cnt[pl.ds(e * 16, 16)] = zero_i
            vsum[pl.ds(e * 16, 16)] = zero_f
            return 0
        lax.fori_loop(0, 256, zero_body, 0)

        def p1(j, acc):
            y = jnp.maximum(row_v[pl.ds(j * 16, 16)], 0.0)
            bits = plsc.bitcast(y, jnp.int32)
            e = lax.shift_right_logical(bits, 23)
            pos = e * 16 + lanes
            plsc.addupdate_scatter(cnt, [pos], ones_i)
            plsc.addupdate_scatter(vsum, [pos], y)
            return acc + y
        accy = lax.fori_loop(0, NVEC, p1, zero_f)
        batch_sum = jnp.sum(accy)

        sufc[pl.ds(256 * 16, 16)] = zero_i
        sufs[pl.ds(256 * 16, 16)] = zero_f

        def sscan(k, carry):
            e = 255 - k
            cacc, sacc = carry
            cacc = cacc + cnt[pl.ds(e * 16, 16)]
            sacc = sacc + vsum[pl.ds(e * 16, 16)]
            sufc[pl.ds(e * 16, 16)] = cacc
            sufs[pl.ds(e * 16, 16)] = sacc
            return (cacc, sacc)
        lax.fori_loop(0, 256, sscan, (zero_i, zero_f))

        def bs(_, lohi):
            lo, hi = lohi
            mid = lax.div(lo + hi, jnp.int32(2))
            good = jnp.sum(sufc[pl.ds(mid * 16, 16)]) >= ks
            return (jnp.where(good, mid, lo), jnp.where(good, hi, mid))
        E, _hi = lax.fori_loop(0, 8, bs, (jnp.int32(0), jnp.int32(256)))
        n_above = jnp.sum(sufc[pl.ds((E + 1) * 16, 16)])
        s_above = jnp.sum(sufs[pl.ds((E + 1) * 16, 16)])
        r_rank = ks - n_above

        def p2(j, off):
            y = jnp.maximum(row_v[pl.ds(j * 16, 16)], 0.0)
            bits = plsc.bitcast(y, jnp.int32)
            msk = lax.shift_right_logical(bits, 23) == E
            pos = off + plsc.cumsum(msk.astype(jnp.int32)) - 1
            plsc.store_scatter(row_v, [pos], y, mask=msk)
            return off + plsc.all_reduce_population_count(msk)
        offv = lax.fori_loop(0, NVEC, p2, zero_i)
        m = jnp.max(offv)
        nv = lax.div(m + 15, jnp.int32(16))
        plsc.store_scatter(row_v, [m + lanes], zero_f,
                           mask=(m + lanes) < nv * 16)

        def bit_body(k, t):
            cand = t | lax.shift_left(jnp.int32(1), 22 - k)

            def cloop(q, a):
                v = plsc.bitcast(row_v[pl.ds(q * 16, 16)], jnp.int32)
                return a + (v >= cand).astype(jnp.int32)
            c = jnp.sum(lax.fori_loop(0, nv, cloop, zero_i))
            return jnp.where(c >= r_rank, cand, t)
        t_bits = lax.fori_loop(0, 23, bit_body, lax.shift_left(E, 23))

        def gtloop(q, carry):
            ac, asum = carry
            v = row_v[pl.ds(q * 16, 16)]
            g = plsc.bitcast(v, jnp.int32) > t_bits
            return (ac + g.astype(jnp.int32), asum + jnp.where(g, v, 0.0))
        gcnt, gsum = lax.fori_loop(0, nv, gtloop, (zero_i, zero_f))
        n_gt = n_above + jnp.sum(gcnt)
        s_gt = s_above + jnp.sum(gsum)
        t_vec = plsc.bitcast(jnp.full((16,), t_bits, jnp.int32), jnp.float32)
        topk = s_gt + (ks - n_gt).astype(jnp.float32) * t_vec
        scale_vec = batch_sum / topk
        plsc.store_scatter(scv, [jnp.full((16,), i, jnp.int32)], scale_vec,
                           mask=lanes == 0)
        return 0

    lax.fori_loop(0, ROWS_PER_W, row_body, 0)
    pltpu.sync_copy(scv, out_hbm.at[wid])


def _sc_scales(feature, ks2):
    mesh = plsc.VectorSubcoreMesh(core_axis_name="c", subcore_axis_name="s")
    f = pl.kernel(
        _sc_scale_body,
        jax.ShapeDtypeStruct((NW, 16), jnp.float32),
        mesh=mesh,
        compiler_params=pltpu.CompilerParams(needs_layout_passes=False),
        scratch_types=[
            pltpu.VMEM((C,), jnp.float32),
            pltpu.VMEM((4096,), jnp.int32),
            pltpu.VMEM((4096,), jnp.float32),
            pltpu.VMEM((4112,), jnp.int32),
            pltpu.VMEM((4112,), jnp.float32),
            pltpu.VMEM((16,), jnp.int32),
            pltpu.VMEM((16,), jnp.float32),
        ],
    )
    return f(feature, ks2)


BN = 256
BC = 4096


def _mm_body(x_ref, w_ref, o_ref):
    j = pl.program_id(1)

    @pl.when(j == 0)
    def _():
        o_ref[...] = jnp.zeros_like(o_ref)

    o_ref[...] += lax.dot_general(
        x_ref[...], w_ref[...],
        dimension_numbers=(((1,), (1,)), ((), ())),
        preferred_element_type=jnp.float32,
    )


def _mm(feature, fc_w_pad):
    return pl.pallas_call(
        _mm_body,
        grid=(N_PAD // BN, C // BC),
        in_specs=[
            pl.BlockSpec((B, BC), lambda i, j: (0, j)),
            pl.BlockSpec((BN, BC), lambda i, j: (i, j)),
        ],
        out_specs=pl.BlockSpec((B, BN), lambda i, j: (0, i)),
        out_shape=jax.ShapeDtypeStruct((B, N_PAD), jnp.float32),
    )(feature, fc_w_pad)


def _comb_body(r_ref, s_ref, b_ref, o_ref):
    o_ref[...] = r_ref[...] * jnp.exp(s_ref[...]) + b_ref[...]


def _combine(raw, scale, fc_b_pad):
    return pl.pallas_call(
        _comb_body,
        out_shape=jax.ShapeDtypeStruct((B, N_PAD), jnp.float32),
    )(raw, scale, fc_b_pad)


def kernel(feature, percentiles, fc_w, fc_b):
    n_classes = fc_w.shape[0]
    ks = C - jnp.round(C * percentiles / 100.0).astype(jnp.int32)
    ks2 = jnp.pad(ks.reshape(NW, ROWS_PER_W), ((0, 0), (0, 16 - ROWS_PER_W)))
    scale = _sc_scales(feature, ks2)[:, :ROWS_PER_W].reshape(B, 1)
    w_pad = jnp.pad(fc_w, ((0, N_PAD - n_classes), (0, 0)))
    b_pad = jnp.pad(fc_b, (0, N_PAD - n_classes)).reshape(1, N_PAD)
    raw = _mm(feature, w_pad)
    out = _combine(raw, scale, b_pad)
    return out[:, :n_classes]


# speedup vs baseline: 1.5039x; 1.5039x over previous
import jax
import jax.numpy as jnp
from jax import lax
from jax.experimental import pallas as pl
from jax.experimental.pallas import tpu as pltpu
from jax.experimental.pallas import tpu_sc as plsc

B = 128
C = 32768
N_PAD = 1024

NC = 2
NW = 32
ROWS_PER_W = B // NW
NVEC = C // 16


NB = 1024
SHIFT = 21
REM = SHIFT


def _sc_scale_body(feature_hbm, ks_hbm, out_hbm, row_v, cnt, sufc, ksv, scv):
    wid = lax.axis_index("s") * NC + lax.axis_index("c")
    lanes = lax.iota(jnp.int32, 16)
    ones_i = jnp.ones((16,), jnp.int32)
    zero_i = jnp.zeros((16,), jnp.int32)
    zero_f = jnp.zeros((16,), jnp.float32)
    pltpu.sync_copy(ks_hbm.at[wid], ksv)
    ks_all = ksv[...]

    def zero_body(k, unused):
        for u in range(8):
            cnt[pl.ds((k * 8 + u) * 16, 16)] = zero_i
        return 0
    lax.fori_loop(0, NB // 8, zero_body, 0)

    def row_body(i, _):
        r = wid * ROWS_PER_W + i
        pltpu.sync_copy(feature_hbm.at[r], row_v)
        ks = jnp.sum(jnp.where(lanes == i, ks_all, jnp.int32(0)))

        def p1(j, acc):
            ys = []
            for u in range(8):
                y = jnp.maximum(row_v[pl.ds(j * 128 + u * 16, 16)], 0.0)
                bits = plsc.bitcast(y, jnp.int32)
                pos = lax.shift_right_logical(bits, SHIFT) * 16 + lanes
                plsc.addupdate_scatter(cnt, [pos], ones_i)
                ys.append(y)
            s01 = (ys[0] + ys[1]) + (ys[2] + ys[3])
            s23 = (ys[4] + ys[5]) + (ys[6] + ys[7])
            return acc + (s01 + s23)
        accy = lax.fori_loop(0, NVEC // 8, p1, zero_f)
        batch_sum = jnp.sum(accy)

        sufc[pl.ds(NB * 16, 16)] = zero_i

        def sscan(k, cacc):
            for u in range(4):
                e = NB - 1 - (k * 4 + u)
                cacc = cacc + cnt[pl.ds(e * 16, 16)]
                cnt[pl.ds(e * 16, 16)] = zero_i
                sufc[pl.ds(e * 16, 16)] = cacc
            return cacc
        lax.fori_loop(0, NB // 4, sscan, zero_i)

        def bs(_, lohi):
            lo, hi = lohi
            mid = lax.div(lo + hi, jnp.int32(2))
            good = jnp.sum(sufc[pl.ds(mid * 16, 16)]) >= ks
            return (jnp.where(good, mid, lo), jnp.where(good, hi, mid))
        E, _hi = lax.fori_loop(0, 10, bs, (jnp.int32(0), jnp.int32(NB)))
        n_above = jnp.sum(sufc[pl.ds((E + 1) * 16, 16)])
        r_rank = ks - n_above
        thr_hi = lax.shift_left(E + 1, SHIFT)

        def p2(j, carry):
            off, sa = carry
            for u in range(4):
                y = jnp.maximum(row_v[pl.ds(j * 64 + u * 16, 16)], 0.0)
                bits = plsc.bitcast(y, jnp.int32)
                msk = lax.shift_right_logical(bits, SHIFT) == E
                pos = off + plsc.cumsum(msk.astype(jnp.int32)) - 1
                plsc.store_scatter(row_v, [pos], y, mask=msk)
                off = off + plsc.all_reduce_population_count(msk)
                sa = sa + jnp.where(bits >= thr_hi, y, 0.0)
            return (off, sa)
        offv, sav = lax.fori_loop(0, NVEC // 4, p2, (zero_i, zero_f))
        m = jnp.max(offv)
        s_above = jnp.sum(sav)
        nv2 = lax.div(m + 31, jnp.int32(32))
        plsc.store_scatter(row_v, [m + lanes], zero_f,
                           mask=(m + lanes) < nv2 * 32)
        plsc.store_scatter(row_v, [m + 16 + lanes], zero_f,
                           mask=(m + 16 + lanes) < nv2 * 32)

        def bit_body(k, t):
            cand = t | lax.shift_left(jnp.int32(1), REM - 1 - k)

            def cloop(q, a):
                v0 = plsc.bitcast(row_v[pl.ds(q * 32, 16)], jnp.int32)
                v1 = plsc.bitcast(row_v[pl.ds(q * 32 + 16, 16)], jnp.int32)
                return (a + (v0 >= cand).astype(jnp.int32)
                        + (v1 >= cand).astype(jnp.int32))
            c = jnp.sum(lax.fori_loop(0, nv2, cloop, zero_i))
            return jnp.where(c >= r_rank, cand, t)
        t_bits = lax.fori_loop(0, REM, bit_body, lax.shift_left(E, SHIFT))

        def gtloop(q, carry):
            ac, asum = carry
            v0 = row_v[pl.ds(q * 32, 16)]
            v1 = row_v[pl.ds(q * 32 + 16, 16)]
            g0 = plsc.bitcast(v0, jnp.int32) > t_bits
            g1 = plsc.bitcast(v1, jnp.int32) > t_bits
            return (ac + g0.astype(jnp.int32) + g1.astype(jnp.int32),
                    asum + jnp.where(g0, v0, 0.0) + jnp.where(g1, v1, 0.0))
        gcnt, gsum = lax.fori_loop(0, nv2, gtloop, (zero_i, zero_f))
        n_gt = n_above + jnp.sum(gcnt)
        s_gt = s_above + jnp.sum(gsum)
        t_vec = plsc.bitcast(jnp.full((16,), t_bits, jnp.int32), jnp.float32)
        topk = s_gt + (ks - n_gt).astype(jnp.float32) * t_vec
        scale_vec = batch_sum / topk
        plsc.store_scatter(scv, [jnp.full((16,), i, jnp.int32)], scale_vec,
                           mask=lanes == 0)
        return 0

    lax.fori_loop(0, ROWS_PER_W, row_body, 0)
    pltpu.sync_copy(scv, out_hbm.at[wid])


def _sc_scales(feature, ks2):
    mesh = plsc.VectorSubcoreMesh(core_axis_name="c", subcore_axis_name="s")
    f = pl.kernel(
        _sc_scale_body,
        jax.ShapeDtypeStruct((NW, 16), jnp.float32),
        mesh=mesh,
        compiler_params=pltpu.CompilerParams(needs_layout_passes=False),
        scratch_types=[
            pltpu.VMEM((C,), jnp.float32),
            pltpu.VMEM((NB * 16,), jnp.int32),
            pltpu.VMEM(((NB + 1) * 16,), jnp.int32),
            pltpu.VMEM((16,), jnp.int32),
            pltpu.VMEM((16,), jnp.float32),
        ],
    )
    return f(feature, ks2)


BN = 256
BC = 4096


def _mm_body(x_ref, w_ref, o_ref):
    j = pl.program_id(1)

    @pl.when(j == 0)
    def _():
        o_ref[...] = jnp.zeros_like(o_ref)

    o_ref[...] += lax.dot_general(
        x_ref[...], w_ref[...],
        dimension_numbers=(((1,), (1,)), ((), ())),
        preferred_element_type=jnp.float32,
    )


def _mm(feature, fc_w_pad):
    return pl.pallas_call(
        _mm_body,
        grid=(N_PAD // BN, C // BC),
        in_specs=[
            pl.BlockSpec((B, BC), lambda i, j: (0, j)),
            pl.BlockSpec((BN, BC), lambda i, j: (i, j)),
        ],
        out_specs=pl.BlockSpec((B, BN), lambda i, j: (0, i)),
        out_shape=jax.ShapeDtypeStruct((B, N_PAD), jnp.float32),
    )(feature, fc_w_pad)


def _comb_body(r_ref, s_ref, b_ref, o_ref):
    o_ref[...] = r_ref[...] * jnp.exp(s_ref[...]) + b_ref[...]


def _combine(raw, scale, fc_b_pad):
    return pl.pallas_call(
        _comb_body,
        out_shape=jax.ShapeDtypeStruct((B, N_PAD), jnp.float32),
    )(raw, scale, fc_b_pad)


def kernel(feature, percentiles, fc_w, fc_b):
    n_classes = fc_w.shape[0]
    ks = C - jnp.round(C * percentiles / 100.0).astype(jnp.int32)
    ks2 = jnp.pad(ks.reshape(NW, ROWS_PER_W), ((0, 0), (0, 16 - ROWS_PER_W)))
    scale = _sc_scales(feature, ks2)[:, :ROWS_PER_W].reshape(B, 1)
    w_pad = jnp.pad(fc_w, ((0, N_PAD - n_classes), (0, 0)))
    b_pad = jnp.pad(fc_b, (0, N_PAD - n_classes)).reshape(1, N_PAD)
    raw = _mm(feature, w_pad)
    out = _combine(raw, scale, b_pad)
    return out[:, :n_classes]


# speedup vs baseline: 2.1328x; 1.4182x over previous
import jax
import jax.numpy as jnp
from jax import lax
from jax.experimental import pallas as pl
from jax.experimental.pallas import tpu as pltpu
from jax.experimental.pallas import tpu_sc as plsc

B = 128
C = 32768
N_PAD = 1024

NC = 2
NW = 32
ROWS_PER_W = B // NW
NVEC = C // 16


NB = 1024
SHIFT = 21
REM = SHIFT


def _sc_scale_body(feature_hbm, ks_hbm, out_hbm, row_v, cnt, sufc, ksv, scv):
    wid = lax.axis_index("s") * NC + lax.axis_index("c")
    lanes = lax.iota(jnp.int32, 16)
    ones_i = jnp.ones((16,), jnp.int32)
    zero_i = jnp.zeros((16,), jnp.int32)
    zero_f = jnp.zeros((16,), jnp.float32)
    pltpu.sync_copy(ks_hbm.at[wid], ksv)
    ks_all = ksv[...]

    @plsc.parallel_loop(0, NB, 1, unroll=8)
    def _zero(k):
        cnt[pl.ds(k * 16, 16)] = zero_i

    def row_body(i, _):
        r = wid * ROWS_PER_W + i
        pltpu.sync_copy(feature_hbm.at[r], row_v)
        ks = jnp.sum(jnp.where(lanes == i, ks_all, jnp.int32(0)))

        @plsc.parallel_loop(0, NVEC, 1, unroll=8, carry=zero_f)
        def accy(j, acc):
            y = jnp.maximum(row_v[pl.ds(j * 16, 16)], 0.0)
            bits = plsc.bitcast(y, jnp.int32)
            pos = lax.shift_right_logical(bits, SHIFT) * 16 + lanes
            plsc.addupdate_scatter(cnt, [pos], ones_i)
            return acc + y
        batch_sum = jnp.sum(accy)

        sufc[pl.ds(NB * 16, 16)] = zero_i

        @plsc.parallel_loop(0, NB, 1, unroll=8, carry=zero_i)
        def _suf(k, cacc):
            e = NB - 1 - k
            cacc = cacc + cnt[pl.ds(e * 16, 16)]
            cnt[pl.ds(e * 16, 16)] = zero_i
            sufc[pl.ds(e * 16, 16)] = cacc
            return cacc

        def bs(_, lohi):
            lo, hi = lohi
            mid = lax.div(lo + hi, jnp.int32(2))
            good = jnp.sum(sufc[pl.ds(mid * 16, 16)]) >= ks
            return (jnp.where(good, mid, lo), jnp.where(good, hi, mid))
        E, _hi = lax.fori_loop(0, 10, bs, (jnp.int32(0), jnp.int32(NB)))
        n_above = jnp.sum(sufc[pl.ds((E + 1) * 16, 16)])
        r_rank = ks - n_above
        thr_hi = lax.shift_left(E + 1, SHIFT)

        @plsc.parallel_loop(0, NVEC, 1, unroll=8, carry=(zero_i, zero_f))
        def _p2(j, carry):
            off, sa = carry
            y = jnp.maximum(row_v[pl.ds(j * 16, 16)], 0.0)
            bits = plsc.bitcast(y, jnp.int32)
            msk = lax.shift_right_logical(bits, SHIFT) == E
            pos = off + plsc.cumsum(msk.astype(jnp.int32)) - 1
            plsc.store_scatter(row_v, [pos], y, mask=msk)
            off = off + plsc.all_reduce_population_count(msk)
            sa = sa + jnp.where(bits >= thr_hi, y, 0.0)
            return (off, sa)
        offv, sav = _p2
        m = jnp.max(offv)
        s_above = jnp.sum(sav)
        nv2 = lax.div(m + 31, jnp.int32(32))
        plsc.store_scatter(row_v, [m + lanes], zero_f,
                           mask=(m + lanes) < nv2 * 32)
        plsc.store_scatter(row_v, [m + 16 + lanes], zero_f,
                           mask=(m + 16 + lanes) < nv2 * 32)

        def bit_body(k, t):
            cand = t | lax.shift_left(jnp.int32(1), REM - 1 - k)

            @plsc.parallel_loop(0, nv2, 1, unroll=4, carry=zero_i)
            def _cnt(q, a):
                v0 = plsc.bitcast(row_v[pl.ds(q * 32, 16)], jnp.int32)
                v1 = plsc.bitcast(row_v[pl.ds(q * 32 + 16, 16)], jnp.int32)
                return (a + (v0 >= cand).astype(jnp.int32)
                        + (v1 >= cand).astype(jnp.int32))
            c = jnp.sum(_cnt)
            return jnp.where(c >= r_rank, cand, t)
        t_bits = lax.fori_loop(0, REM, bit_body, lax.shift_left(E, SHIFT))

        @plsc.parallel_loop(0, nv2, 1, unroll=4, carry=(zero_i, zero_f))
        def _gt(q, carry):
            ac, asum = carry
            v0 = row_v[pl.ds(q * 32, 16)]
            v1 = row_v[pl.ds(q * 32 + 16, 16)]
            g0 = plsc.bitcast(v0, jnp.int32) > t_bits
            g1 = plsc.bitcast(v1, jnp.int32) > t_bits
            return (ac + g0.astype(jnp.int32) + g1.astype(jnp.int32),
                    asum + jnp.where(g0, v0, 0.0) + jnp.where(g1, v1, 0.0))
        gcnt, gsum = _gt
        n_gt = n_above + jnp.sum(gcnt)
        s_gt = s_above + jnp.sum(gsum)
        t_vec = plsc.bitcast(jnp.full((16,), t_bits, jnp.int32), jnp.float32)
        topk = s_gt + (ks - n_gt).astype(jnp.float32) * t_vec
        scale_vec = batch_sum / topk
        plsc.store_scatter(scv, [jnp.full((16,), i, jnp.int32)], scale_vec,
                           mask=lanes == 0)
        return 0

    lax.fori_loop(0, ROWS_PER_W, row_body, 0)
    pltpu.sync_copy(scv, out_hbm.at[wid])


def _sc_scales(feature, ks2):
    mesh = plsc.VectorSubcoreMesh(core_axis_name="c", subcore_axis_name="s")
    f = pl.kernel(
        _sc_scale_body,
        jax.ShapeDtypeStruct((NW, 16), jnp.float32),
        mesh=mesh,
        compiler_params=pltpu.CompilerParams(needs_layout_passes=False),
        scratch_types=[
            pltpu.VMEM((C,), jnp.float32),
            pltpu.VMEM((NB * 16,), jnp.int32),
            pltpu.VMEM(((NB + 1) * 16,), jnp.int32),
            pltpu.VMEM((16,), jnp.int32),
            pltpu.VMEM((16,), jnp.float32),
        ],
    )
    return f(feature, ks2)


BN = 256
BC = 4096


def _mm_body(x_ref, w_ref, o_ref):
    j = pl.program_id(1)

    @pl.when(j == 0)
    def _():
        o_ref[...] = jnp.zeros_like(o_ref)

    o_ref[...] += lax.dot_general(
        x_ref[...], w_ref[...],
        dimension_numbers=(((1,), (1,)), ((), ())),
        preferred_element_type=jnp.float32,
    )


def _mm(feature, fc_w_pad):
    return pl.pallas_call(
        _mm_body,
        grid=(N_PAD // BN, C // BC),
        in_specs=[
            pl.BlockSpec((B, BC), lambda i, j: (0, j)),
            pl.BlockSpec((BN, BC), lambda i, j: (i, j)),
        ],
        out_specs=pl.BlockSpec((B, BN), lambda i, j: (0, i)),
        out_shape=jax.ShapeDtypeStruct((B, N_PAD), jnp.float32),
    )(feature, fc_w_pad)


def _comb_body(r_ref, s_ref, b_ref, o_ref):
    o_ref[...] = r_ref[...] * jnp.exp(s_ref[...]) + b_ref[...]


def _combine(raw, scale, fc_b_pad):
    return pl.pallas_call(
        _comb_body,
        out_shape=jax.ShapeDtypeStruct((B, N_PAD), jnp.float32),
    )(raw, scale, fc_b_pad)


def kernel(feature, percentiles, fc_w, fc_b):
    n_classes = fc_w.shape[0]
    ks = C - jnp.round(C * percentiles / 100.0).astype(jnp.int32)
    ks2 = jnp.pad(ks.reshape(NW, ROWS_PER_W), ((0, 0), (0, 16 - ROWS_PER_W)))
    scale = _sc_scales(feature, ks2)[:, :ROWS_PER_W].reshape(B, 1)
    w_pad = jnp.pad(fc_w, ((0, N_PAD - n_classes), (0, 0)))
    b_pad = jnp.pad(fc_b, (0, N_PAD - n_classes)).reshape(1, N_PAD)
    raw = _mm(feature, w_pad)
    out = _combine(raw, scale, b_pad)
    return out[:, :n_classes]
